# baseline (device time: 18566 ns/iter reference)
import functools

import jax
import jax.numpy as jnp
from jax import lax
from jax.experimental import pallas as pl
from jax.experimental.pallas import tpu as pltpu

M = 1024
N = 512
H = M // 2
CH = 128
NCH = H // CH


def kernel(x, dest):
    dest2d = dest.reshape(1, M)

    def body(x_ref, dest_ref, out_ref, xsend_ref, xbuf_ref, ybuf_ref,
             dpeer_ref, dsems, xs_send, xs_recv, ys_send, ys_recv):
        my_x = lax.axis_index("x")
        my_y = lax.axis_index("y")
        xpeer = (1 - my_x, my_y)
        ypeer = (my_x, 1 - my_y)

        xsend_ref[...] = x_ref[...].astype(jnp.bfloat16)

        barrier_sem = pltpu.get_barrier_semaphore()
        for nbr in (xpeer, ypeer):
            pl.semaphore_signal(barrier_sem, inc=1, device_id=nbr,
                                device_id_type=pl.DeviceIdType.MESH)
        pl.semaphore_wait(barrier_sem, 2)

        rdma_d = pltpu.make_async_remote_copy(
            src_ref=dest_ref, dst_ref=dpeer_ref,
            send_sem=dsems.at[0], recv_sem=dsems.at[1],
            device_id=xpeer, device_id_type=pl.DeviceIdType.MESH)
        rdma_d.start()

        half0 = my_y * H
        x_rdmas = []
        for k in range(NCH):
            r = pltpu.make_async_remote_copy(
                src_ref=xsend_ref.at[pl.ds(half0 + k * CH, CH)],
                dst_ref=xbuf_ref.at[pl.ds(k * CH, CH)],
                send_sem=xs_send.at[k], recv_sem=xs_recv.at[k],
                device_id=xpeer, device_id_type=pl.DeviceIdType.MESH)
            r.start()
            x_rdmas.append(r)

        iota_i = lax.broadcasted_iota(jnp.int32, (M, M), 0)
        iota_j = lax.broadcasted_iota(jnp.int32, (M, M), 1)
        tri = (iota_i <= iota_j).astype(jnp.float32)

        rdma_d.wait()

        dl = dest_ref[...]
        dp = dpeer_ref[...]
        ml = (dl == my_x)
        mp = (dp == my_x)
        csl = jnp.dot(ml.astype(jnp.float32), tri,
                      preferred_element_type=jnp.float32).astype(jnp.int32)
        csp = jnp.dot(mp.astype(jnp.float32), tri,
                      preferred_element_type=jnp.float32).astype(jnp.int32)
        cl = csl[0, M - 1]
        cp = csp[0, M - 1]
        off_l = jnp.where(my_x == 0, 0, cp)
        off_p = jnp.where(my_x == 0, cl, 0)
        posl = off_l + csl - 1
        posp = off_p + csp - 1

        def arrival_order(v):
            swapped = jnp.concatenate([v[:, H:], v[:, :H]], axis=1)
            return jnp.where(my_y == 0, v, swapped)

        posp_a = arrival_order(posp)
        mp_a = arrival_order(mp.astype(jnp.int32))

        def fwd(k):
            x_rdmas[k].wait_recv()
            r = pltpu.make_async_remote_copy(
                src_ref=xbuf_ref.at[pl.ds(k * CH, CH)],
                dst_ref=ybuf_ref.at[pl.ds(k * CH, CH)],
                send_sem=ys_send.at[k], recv_sem=ys_recv.at[k],
                device_id=ypeer, device_id_type=pl.DeviceIdType.MESH)
            r.start()
            y_rdmas.append(r)

        y_rdmas = []
        fwd(0)
        p_l = ((iota_i == posl) & ml).astype(jnp.bfloat16)
        fwd(1)
        p_p = ((iota_i == posp_a) & (mp_a > 0)).astype(jnp.bfloat16)
        fwd(2)
        acc = jnp.dot(p_l, xsend_ref[...], preferred_element_type=jnp.float32)
        fwd(3)
        acc = acc + jnp.dot(p_p[:, :H], xbuf_ref[...],
                            preferred_element_type=jnp.float32)
        for k in range(NCH):
            y_rdmas[k].wait_recv()
        acc = acc + jnp.dot(p_p[:, H:], ybuf_ref[...],
                            preferred_element_type=jnp.float32)

        out_ref[...] = acc.astype(jnp.bfloat16)

        for k in range(NCH):
            x_rdmas[k].wait_send()
            y_rdmas[k].wait_send()

        @functools.partial(pl.run_scoped, sem2=pltpu.SemaphoreType.REGULAR)
        def _(sem2):
            for nbr in (xpeer, ypeer):
                pl.semaphore_signal(sem2, inc=1, device_id=nbr,
                                    device_id_type=pl.DeviceIdType.MESH)
            pl.semaphore_wait(sem2, 2)

    return pl.pallas_call(
        body,
        out_shape=jax.ShapeDtypeStruct((M, N), jnp.bfloat16),
        in_specs=[pl.BlockSpec(memory_space=pltpu.VMEM),
                  pl.BlockSpec(memory_space=pltpu.VMEM)],
        out_specs=pl.BlockSpec(memory_space=pltpu.VMEM),
        scratch_shapes=[
            pltpu.VMEM((M, N), jnp.bfloat16),
            pltpu.VMEM((H, N), jnp.bfloat16),
            pltpu.VMEM((H, N), jnp.bfloat16),
            pltpu.VMEM((1, M), jnp.int32),
            pltpu.SemaphoreType.DMA((2,)),
            pltpu.SemaphoreType.DMA((NCH,)),
            pltpu.SemaphoreType.DMA((NCH,)),
            pltpu.SemaphoreType.DMA((NCH,)),
            pltpu.SemaphoreType.DMA((NCH,)),
        ],
        compiler_params=pltpu.CompilerParams(collective_id=0),
    )(x, dest2d)


# device time: 18247 ns/iter; 1.0175x vs baseline; 1.0175x over previous
import functools

import jax
import jax.numpy as jnp
from jax import lax
from jax.experimental import pallas as pl
from jax.experimental.pallas import tpu as pltpu

M = 1024
N = 512
H = M // 2
CH = 128
NCH = H // CH


def kernel(x, dest):
    dest2d = dest.reshape(1, M)

    def body(x_ref, dest_ref, out_ref, xsend_ref, xbuf_ref, ybuf_ref,
             dpeer_ref, dsems, xs_send, xs_recv, ys_send, ys_recv):
        my_x = lax.axis_index("x")
        my_y = lax.axis_index("y")
        xpeer = (1 - my_x, my_y)
        ypeer = (my_x, 1 - my_y)

        barrier_sem = pltpu.get_barrier_semaphore()
        for nbr in (xpeer, ypeer):
            pl.semaphore_signal(barrier_sem, inc=1, device_id=nbr,
                                device_id_type=pl.DeviceIdType.MESH)
        pl.semaphore_wait(barrier_sem, 2)

        rdma_d = pltpu.make_async_remote_copy(
            src_ref=dest_ref, dst_ref=dpeer_ref,
            send_sem=dsems.at[0], recv_sem=dsems.at[1],
            device_id=xpeer, device_id_type=pl.DeviceIdType.MESH)
        rdma_d.start()

        xsend_ref[...] = x_ref[...].astype(jnp.bfloat16)

        half0 = my_y * H
        x_rdmas = []
        for k in range(NCH):
            r = pltpu.make_async_remote_copy(
                src_ref=xsend_ref.at[pl.ds(half0 + k * CH, CH)],
                dst_ref=xbuf_ref.at[pl.ds(k * CH, CH)],
                send_sem=xs_send.at[k], recv_sem=xs_recv.at[k],
                device_id=xpeer, device_id_type=pl.DeviceIdType.MESH)
            r.start()
            x_rdmas.append(r)

        iota_i = lax.broadcasted_iota(jnp.int32, (M, M), 0)

        rdma_d.wait()

        dl = dest_ref[...]
        dp = dpeer_ref[...]
        ml = (dl == my_x)
        mp = (dp == my_x)

        def cumsum_row(v):
            s = 1
            while s < M:
                v = v + jnp.concatenate(
                    [jnp.zeros((1, s), jnp.int32), v[:, :M - s]], axis=1)
                s *= 2
            return v

        csl = cumsum_row(ml.astype(jnp.int32))
        csp = cumsum_row(mp.astype(jnp.int32))
        cl = csl[0, M - 1]
        cp = csp[0, M - 1]
        off_l = jnp.where(my_x == 0, 0, cp)
        off_p = jnp.where(my_x == 0, cl, 0)
        posl = off_l + csl - 1
        posp = off_p + csp - 1

        def arrival_order(v):
            swapped = jnp.concatenate([v[:, H:], v[:, :H]], axis=1)
            return jnp.where(my_y == 0, v, swapped)

        posp_a = arrival_order(posp)
        mp_a = arrival_order(mp.astype(jnp.int32))

        def fwd(k):
            x_rdmas[k].wait_recv()
            r = pltpu.make_async_remote_copy(
                src_ref=xbuf_ref.at[pl.ds(k * CH, CH)],
                dst_ref=ybuf_ref.at[pl.ds(k * CH, CH)],
                send_sem=ys_send.at[k], recv_sem=ys_recv.at[k],
                device_id=ypeer, device_id_type=pl.DeviceIdType.MESH)
            r.start()
            y_rdmas.append(r)

        y_rdmas = []
        fwd(0)
        p_l = ((iota_i == posl) & ml).astype(jnp.bfloat16)
        fwd(1)
        p_p = ((iota_i == posp_a) & (mp_a > 0)).astype(jnp.bfloat16)
        fwd(2)
        acc = jnp.dot(p_l, xsend_ref[...], preferred_element_type=jnp.float32)
        fwd(3)
        acc = acc + jnp.dot(p_p[:, :H], xbuf_ref[...],
                            preferred_element_type=jnp.float32)
        HH = H // 2
        y_rdmas[0].wait_recv()
        y_rdmas[1].wait_recv()
        acc = acc + jnp.dot(p_p[:, H:H + HH], ybuf_ref[pl.ds(0, HH), :],
                            preferred_element_type=jnp.float32)
        y_rdmas[2].wait_recv()
        y_rdmas[3].wait_recv()
        acc = acc + jnp.dot(p_p[:, H + HH:], ybuf_ref[pl.ds(HH, HH), :],
                            preferred_element_type=jnp.float32)

        out_ref[...] = acc.astype(jnp.bfloat16)

        for k in range(NCH):
            x_rdmas[k].wait_send()
            y_rdmas[k].wait_send()

        @functools.partial(pl.run_scoped, sem2=pltpu.SemaphoreType.REGULAR)
        def _(sem2):
            for nbr in (xpeer, ypeer):
                pl.semaphore_signal(sem2, inc=1, device_id=nbr,
                                    device_id_type=pl.DeviceIdType.MESH)
            pl.semaphore_wait(sem2, 2)

    return pl.pallas_call(
        body,
        out_shape=jax.ShapeDtypeStruct((M, N), jnp.bfloat16),
        in_specs=[pl.BlockSpec(memory_space=pltpu.VMEM),
                  pl.BlockSpec(memory_space=pltpu.VMEM)],
        out_specs=pl.BlockSpec(memory_space=pltpu.VMEM),
        scratch_shapes=[
            pltpu.VMEM((M, N), jnp.bfloat16),
            pltpu.VMEM((H, N), jnp.bfloat16),
            pltpu.VMEM((H, N), jnp.bfloat16),
            pltpu.VMEM((1, M), jnp.int32),
            pltpu.SemaphoreType.DMA((2,)),
            pltpu.SemaphoreType.DMA((NCH,)),
            pltpu.SemaphoreType.DMA((NCH,)),
            pltpu.SemaphoreType.DMA((NCH,)),
            pltpu.SemaphoreType.DMA((NCH,)),
        ],
        compiler_params=pltpu.CompilerParams(collective_id=0),
    )(x, dest2d)


# device time: 16715 ns/iter; 1.1107x vs baseline; 1.0917x over previous
import functools

import jax
import jax.numpy as jnp
from jax import lax
from jax.experimental import pallas as pl
from jax.experimental.pallas import tpu as pltpu

M = 1024
N = 512
H = M // 2
CH = 64
NCH = H // CH


def kernel(x, dest):
    dest2d = dest.reshape(1, M)

    def body(x_ref, dest_ref, out_ref, xsend_ref, xbuf_ref, ybuf_ref,
             dpeer_ref, dsems, xs_send, xs_recv, ys_send, ys_recv):
        my_x = lax.axis_index("x")
        my_y = lax.axis_index("y")
        xpeer = (1 - my_x, my_y)
        ypeer = (my_x, 1 - my_y)

        barrier_sem = pltpu.get_barrier_semaphore()
        for nbr in (xpeer, ypeer):
            pl.semaphore_signal(barrier_sem, inc=1, device_id=nbr,
                                device_id_type=pl.DeviceIdType.MESH)
        pl.semaphore_wait(barrier_sem, 2)

        rdma_d = pltpu.make_async_remote_copy(
            src_ref=dest_ref, dst_ref=dpeer_ref,
            send_sem=dsems.at[0], recv_sem=dsems.at[1],
            device_id=xpeer, device_id_type=pl.DeviceIdType.MESH)
        rdma_d.start()

        xsend_ref[...] = x_ref[...].astype(jnp.bfloat16)

        half0 = my_y * H
        x_rdmas = []
        for k in range(NCH):
            r = pltpu.make_async_remote_copy(
                src_ref=xsend_ref.at[pl.ds(half0 + k * CH, CH)],
                dst_ref=xbuf_ref.at[pl.ds(k * CH, CH)],
                send_sem=xs_send.at[k], recv_sem=xs_recv.at[k],
                device_id=xpeer, device_id_type=pl.DeviceIdType.MESH)
            r.start()
            x_rdmas.append(r)

        iota_i = lax.broadcasted_iota(jnp.int32, (M, M), 0)

        rdma_d.wait()

        dl = dest_ref[...]
        dp = dpeer_ref[...]
        ml = (dl == my_x)
        mp = (dp == my_x)

        def cumsum_row(v):
            s = 1
            while s < M:
                v = v + jnp.concatenate(
                    [jnp.zeros((1, s), jnp.int32), v[:, :M - s]], axis=1)
                s *= 2
            return v

        csl = cumsum_row(ml.astype(jnp.int32))
        csp = cumsum_row(mp.astype(jnp.int32))
        cl = csl[0, M - 1]
        cp = csp[0, M - 1]
        off_l = jnp.where(my_x == 0, 0, cp)
        off_p = jnp.where(my_x == 0, cl, 0)
        posl = off_l + csl - 1
        posp = off_p + csp - 1

        def arrival_order(v):
            swapped = jnp.concatenate([v[:, H:], v[:, :H]], axis=1)
            return jnp.where(my_y == 0, v, swapped)

        posp_a = arrival_order(posp)
        mp_a = arrival_order(mp.astype(jnp.int32))

        def fwd(k):
            x_rdmas[k].wait_recv()
            r = pltpu.make_async_remote_copy(
                src_ref=xbuf_ref.at[pl.ds(k * CH, CH)],
                dst_ref=ybuf_ref.at[pl.ds(k * CH, CH)],
                send_sem=ys_send.at[k], recv_sem=ys_recv.at[k],
                device_id=ypeer, device_id_type=pl.DeviceIdType.MESH)
            r.start()
            y_rdmas.append(r)

        y_rdmas = []
        fwd(0)
        p_l = ((iota_i == posl) & ml).astype(jnp.bfloat16)
        fwd(1)
        p_p = ((iota_i == posp_a) & (mp_a > 0)).astype(jnp.bfloat16)
        fwd(2)
        fwd(3)
        acc = jnp.dot(p_l, xsend_ref[...], preferred_element_type=jnp.float32)
        for k in range(4, NCH):
            fwd(k)
        acc = acc + jnp.dot(p_p[:, :H], xbuf_ref[...],
                            preferred_element_type=jnp.float32)
        HH = H // 2
        for k in range(NCH // 2):
            y_rdmas[k].wait_recv()
        acc = acc + jnp.dot(p_p[:, H:H + HH], ybuf_ref[pl.ds(0, HH), :],
                            preferred_element_type=jnp.float32)
        for k in range(NCH // 2, NCH):
            y_rdmas[k].wait_recv()
        acc = acc + jnp.dot(p_p[:, H + HH:], ybuf_ref[pl.ds(HH, HH), :],
                            preferred_element_type=jnp.float32)

        out_ref[...] = acc.astype(jnp.bfloat16)

        for k in range(NCH):
            x_rdmas[k].wait_send()
            y_rdmas[k].wait_send()


    return pl.pallas_call(
        body,
        out_shape=jax.ShapeDtypeStruct((M, N), jnp.bfloat16),
        in_specs=[pl.BlockSpec(memory_space=pltpu.VMEM),
                  pl.BlockSpec(memory_space=pltpu.VMEM)],
        out_specs=pl.BlockSpec(memory_space=pltpu.VMEM),
        scratch_shapes=[
            pltpu.VMEM((M, N), jnp.bfloat16),
            pltpu.VMEM((H, N), jnp.bfloat16),
            pltpu.VMEM((H, N), jnp.bfloat16),
            pltpu.VMEM((1, M), jnp.int32),
            pltpu.SemaphoreType.DMA((2,)),
            pltpu.SemaphoreType.DMA((NCH,)),
            pltpu.SemaphoreType.DMA((NCH,)),
            pltpu.SemaphoreType.DMA((NCH,)),
            pltpu.SemaphoreType.DMA((NCH,)),
        ],
        compiler_params=pltpu.CompilerParams(collective_id=0),
    )(x, dest2d)
